# Initial kernel scaffold; baseline (speedup 1.0000x reference)
import jax
import jax.numpy as jnp
from jax.experimental import pallas as pl


def kernel(x, edge_index, W1, b1, W2, b2):
    raise NotImplementedError("write your pallas kernel here")


# speedup vs baseline: 27.4705x; 27.4705x over previous
import functools

import jax
import jax.numpy as jnp
from jax import lax
from jax.experimental import pallas as pl
from jax.experimental.pallas import tpu as pltpu
from jax.experimental.pallas import tpu_sc as plsc

N = 10000
E = 320000
D = 128
NC = 2
NS = 16
LANE = 128
C = 158
EP = NS * C * LANE
NP = 10240
ROWS_PER_TILE = NP // NS
DH = D // NC


_MESH = plsc.VectorSubcoreMesh(core_axis_name="c", subcore_axis_name="s")


@functools.partial(
    pl.kernel,
    out_type=jax.ShapeDtypeStruct((NC * NP,), jnp.float32),
    mesh=_MESH,
    scratch_types=[
        pltpu.VMEM((C // 2, LANE), jnp.int32),
        pltpu.VMEM((LANE,), jnp.float32),
        pltpu.VMEM_SHARED((NP,), jnp.float32),
    ],
)
def _deg(dst_hbm, zeros_hbm, out_hbm, idx_v, ones_v, acc_s):
    cid = lax.axis_index("c")
    sid = lax.axis_index("s")
    wid = sid * NC + cid
    pltpu.sync_copy(dst_hbm.at[wid], idx_v)
    for i in range(LANE // 16):
        ones_v[pl.ds(i * 16, 16)] = jnp.ones((16,), jnp.float32)

    @pl.when(sid == 0)
    def _():
        pltpu.sync_copy(zeros_hbm, acc_s)

    plsc.subcore_barrier()

    def body(k, carry):
        pltpu.sync_copy(ones_v, acc_s.at[idx_v.at[k]], add=True)
        return carry

    lax.fori_loop(0, C // 2, body, 0)
    plsc.subcore_barrier()
    base = sid * ROWS_PER_TILE
    pltpu.sync_copy(acc_s.at[pl.ds(base, ROWS_PER_TILE)],
                    out_hbm.at[pl.ds(cid * NP + base, ROWS_PER_TILE)])


@functools.partial(
    pl.kernel,
    out_type=jax.ShapeDtypeStruct((NC, NP, DH), jnp.float32),
    mesh=_MESH,
    scratch_types=[
        pltpu.VMEM((C, LANE), jnp.int32),
        pltpu.VMEM((C, LANE), jnp.int32),
        pltpu.VMEM((2, LANE, DH), jnp.float32),
        pltpu.VMEM_SHARED((NP, DH), jnp.float32),
        pltpu.SemaphoreType.DMA,
        pltpu.SemaphoreType.DMA,
    ],
    compiler_params=pltpu.CompilerParams(use_tc_tiling_on_sc=False),
)
def _gather_scatter(y_hbm, src_hbm, dst_hbm, zeros_hbm, out_hbm,
                    src_v, dst_v, buf_v, acc_s, sem0, sem1):
    cid = lax.axis_index("c")
    sid = lax.axis_index("s")
    pltpu.sync_copy(src_hbm.at[cid, sid], src_v)
    pltpu.sync_copy(dst_hbm.at[sid], dst_v)

    @pl.when(sid == 0)
    def _():
        pltpu.sync_copy(zeros_hbm, acc_s)

    plsc.subcore_barrier()

    sems = (sem0, sem1)
    for b in range(2):
        pltpu.async_copy(y_hbm.at[src_v.at[b]], buf_v.at[b], sems[b])

    def body(j, carry):
        for b in range(2):
            k = j * 2 + b
            pltpu.make_async_copy(y_hbm.at[src_v.at[k]], buf_v.at[b],
                                  sems[b]).wait()
            pltpu.sync_copy(buf_v.at[b], acc_s.at[dst_v.at[k]], add=True)
            kn = k + 2

            @pl.when(kn < C)
            def _():
                pltpu.async_copy(y_hbm.at[src_v.at[kn]], buf_v.at[b], sems[b])

        return carry

    lax.fori_loop(0, C // 2, body, 0)
    plsc.subcore_barrier()
    base = sid * ROWS_PER_TILE
    pltpu.sync_copy(acc_s.at[pl.ds(base, ROWS_PER_TILE)],
                    out_hbm.at[cid, pl.ds(base, ROWS_PER_TILE)])


def _tc1_body(x_ref, w_ref, dh_ref, y_ref):
    dinv = lax.rsqrt(1.0 + dh_ref[0, :] + dh_ref[1, :])
    xw = jnp.dot(x_ref[...], w_ref[...], preferred_element_type=jnp.float32)
    y = xw * dinv[:, None]
    y_ref[0] = y[:, :DH]
    y_ref[1] = y[:, DH:]


def _tc2_body(s_ref, y_ref, dh_ref, b_ref, w_ref, o_ref):
    dinv = lax.rsqrt(1.0 + dh_ref[0, :] + dh_ref[1, :])
    t = jnp.concatenate([s_ref[0] + y_ref[0], s_ref[1] + y_ref[1]], axis=1)
    h = jax.nn.relu(t * dinv[:, None] + b_ref[...][None, :])
    y2 = jnp.dot(h, w_ref[...], preferred_element_type=jnp.float32)
    y2 = y2 * dinv[:, None]
    o_ref[0] = y2[:, :DH]
    o_ref[1] = y2[:, DH:]


def _tc3_body(s_ref, y_ref, dh_ref, b_ref, o_ref):
    dinv = lax.rsqrt(1.0 + dh_ref[0, :] + dh_ref[1, :])
    t = jnp.concatenate([s_ref[0] + y_ref[0], s_ref[1] + y_ref[1]], axis=1)
    o_ref[...] = jax.nn.relu(t * dinv[:, None] + b_ref[...][None, :])[:N]


_tc1 = pl.pallas_call(
    _tc1_body, out_shape=jax.ShapeDtypeStruct((NC, NP, DH), jnp.float32))
_tc2 = pl.pallas_call(
    _tc2_body, out_shape=jax.ShapeDtypeStruct((NC, NP, DH), jnp.float32))
_tc3 = pl.pallas_call(
    _tc3_body, out_shape=jax.ShapeDtypeStruct((N, D), jnp.float32))


def kernel(x, edge_index, W1, b1, W2, b2):
    ei = edge_index.astype(jnp.int32)
    src, dst = ei[0], ei[1]
    pad = EP - E
    pad_ids = (jnp.arange(pad, dtype=jnp.int32) % (NP - N)) + N
    src_p = jnp.concatenate([src, pad_ids])
    dst_p = jnp.concatenate([dst, pad_ids])
    src_t = src_p.reshape(NS, C, LANE)
    src_t = jnp.stack([src_t, src_t + NP])
    dst_t = dst_p.reshape(NS, C, LANE)
    dst_w = dst_p.reshape(NC * NS, C // 2, LANE)

    x_pad = jnp.zeros((NP, D), jnp.float32).at[:N].set(x.astype(jnp.float32))
    zeros_deg = jnp.zeros((NP,), jnp.float32)
    zeros_col = jnp.zeros((NP, DH), jnp.float32)

    dh = _deg(dst_w, zeros_deg).reshape(NC, NP)
    y1 = _tc1(x_pad, W1, dh)
    s1 = _gather_scatter(y1.reshape(NC * NP, DH), src_t, dst_t, zeros_col)
    y2 = _tc2(s1, y1, dh, b1, W2)
    s2 = _gather_scatter(y2.reshape(NC * NP, DH), src_t, dst_t, zeros_col)
    return _tc3(s2, y2, dh, b2)


# speedup vs baseline: 30.0101x; 1.0924x over previous
import functools

import jax
import jax.numpy as jnp
from jax import lax
from jax.experimental import pallas as pl
from jax.experimental.pallas import tpu as pltpu
from jax.experimental.pallas import tpu_sc as plsc

N = 10000
E = 320000
D = 128
NC = 2
NS = 16
LANE = 128
C = 160
EP = NS * C * LANE
NP = 10240
ROWS_PER_TILE = NP // NS
DH = D // NC


_MESH = plsc.VectorSubcoreMesh(core_axis_name="c", subcore_axis_name="s")


@functools.partial(
    pl.kernel,
    out_type=jax.ShapeDtypeStruct((NC * NP,), jnp.float32),
    mesh=_MESH,
    scratch_types=[
        pltpu.VMEM((C // 2, LANE), jnp.int32),
        pltpu.VMEM((LANE,), jnp.float32),
        pltpu.VMEM((ROWS_PER_TILE,), jnp.float32),
        pltpu.VMEM_SHARED((NP,), jnp.float32),
    ],
)
def _deg(dst_hbm, out_hbm, idx_v, ones_v, zb_v, acc_s):
    cid = lax.axis_index("c")
    sid = lax.axis_index("s")
    wid = sid * NC + cid
    pltpu.sync_copy(dst_hbm.at[wid], idx_v)
    for i in range(LANE // 16):
        ones_v[pl.ds(i * 16, 16)] = jnp.ones((16,), jnp.float32)
    for i in range(ROWS_PER_TILE // 16):
        zb_v[pl.ds(i * 16, 16)] = jnp.zeros((16,), jnp.float32)
    base = sid * ROWS_PER_TILE
    pltpu.sync_copy(zb_v, acc_s.at[pl.ds(base, ROWS_PER_TILE)])
    plsc.subcore_barrier()

    def body(k, carry):
        pltpu.sync_copy(ones_v, acc_s.at[idx_v.at[k]], add=True)
        return carry

    lax.fori_loop(0, C // 2, body, 0)
    plsc.subcore_barrier()
    pltpu.sync_copy(acc_s.at[pl.ds(base, ROWS_PER_TILE)],
                    out_hbm.at[pl.ds(cid * NP + base, ROWS_PER_TILE)])


@functools.partial(
    pl.kernel,
    out_type=jax.ShapeDtypeStruct((NC, NP, DH), jnp.float32),
    mesh=_MESH,
    scratch_types=[
        pltpu.VMEM((C, LANE), jnp.int32),
        pltpu.VMEM((C, LANE), jnp.int32),
        pltpu.VMEM((4, LANE, DH), jnp.float32),
        pltpu.VMEM((LANE, DH), jnp.float32),
        pltpu.VMEM_SHARED((NP, DH), jnp.float32),
        [pltpu.SemaphoreType.DMA] * 4,
        [pltpu.SemaphoreType.DMA] * 4,
    ],
    compiler_params=pltpu.CompilerParams(use_tc_tiling_on_sc=False),
)
def _gather_scatter(y_hbm, src_hbm, dst_hbm, out_hbm,
                    src_v, dst_v, buf_v, zb_v, acc_s, gsems, ssems):
    cid = lax.axis_index("c")
    sid = lax.axis_index("s")
    pltpu.sync_copy(src_hbm.at[cid, sid], src_v)
    pltpu.sync_copy(dst_hbm.at[sid], dst_v)

    def zrow(r, carry):
        row = zb_v.at[r]
        for j in range(DH // 16):
            row[pl.ds(j * 16, 16)] = jnp.zeros((16,), jnp.float32)
        return carry

    lax.fori_loop(0, LANE, zrow, 0)
    base = sid * ROWS_PER_TILE
    for i in range(ROWS_PER_TILE // LANE):
        pltpu.sync_copy(zb_v, acc_s.at[pl.ds(base + i * LANE, LANE)])
    plsc.subcore_barrier()

    for b in range(4):
        pltpu.async_copy(y_hbm.at[src_v.at[b]], buf_v.at[b], gsems[b])

    def body(j, carry):
        for b in range(4):
            k = j * 4 + b
            pltpu.make_async_copy(y_hbm.at[src_v.at[k]], buf_v.at[b],
                                  gsems[b]).wait()
            pltpu.async_copy(buf_v.at[b], acc_s.at[dst_v.at[k]], ssems[b],
                             add=True)
        for b in range(4):
            k = j * 4 + b
            kn = k + 4

            @pl.when(kn < C)
            def _():
                pltpu.make_async_copy(buf_v.at[b], acc_s.at[dst_v.at[k]],
                                      ssems[b]).wait()
                pltpu.async_copy(y_hbm.at[src_v.at[kn]], buf_v.at[b], gsems[b])

        return carry

    lax.fori_loop(0, C // 4, body, 0)
    for b in range(4):
        k = C - 4 + b
        pltpu.make_async_copy(buf_v.at[b], acc_s.at[dst_v.at[k]],
                              ssems[b]).wait()
    plsc.subcore_barrier()
    pltpu.sync_copy(acc_s.at[pl.ds(base, ROWS_PER_TILE)],
                    out_hbm.at[cid, pl.ds(base, ROWS_PER_TILE)])


def _tc1_body(x_ref, w_ref, dh_ref, y_ref):
    dinv = lax.rsqrt(1.0 + dh_ref[0, :] + dh_ref[1, :])
    xw = jnp.dot(x_ref[...], w_ref[...], preferred_element_type=jnp.float32)
    y = xw * dinv[:, None]
    y_ref[0] = y[:, :DH]
    y_ref[1] = y[:, DH:]


def _tc2_body(s_ref, y_ref, dh_ref, b_ref, w_ref, o_ref):
    dinv = lax.rsqrt(1.0 + dh_ref[0, :] + dh_ref[1, :])
    t = jnp.concatenate([s_ref[0] + y_ref[0], s_ref[1] + y_ref[1]], axis=1)
    h = jax.nn.relu(t * dinv[:, None] + b_ref[...][None, :])
    y2 = jnp.dot(h, w_ref[...], preferred_element_type=jnp.float32)
    y2 = y2 * dinv[:, None]
    o_ref[0] = y2[:, :DH]
    o_ref[1] = y2[:, DH:]


def _tc3_body(s_ref, y_ref, dh_ref, b_ref, o_ref):
    dinv = lax.rsqrt(1.0 + dh_ref[0, :] + dh_ref[1, :])
    t = jnp.concatenate([s_ref[0] + y_ref[0], s_ref[1] + y_ref[1]], axis=1)
    o_ref[...] = jax.nn.relu(t * dinv[:, None] + b_ref[...][None, :])[:N]


_tc1 = pl.pallas_call(
    _tc1_body, out_shape=jax.ShapeDtypeStruct((NC, NP, DH), jnp.float32))
_tc2 = pl.pallas_call(
    _tc2_body, out_shape=jax.ShapeDtypeStruct((NC, NP, DH), jnp.float32))
_tc3 = pl.pallas_call(
    _tc3_body, out_shape=jax.ShapeDtypeStruct((N, D), jnp.float32))


def kernel(x, edge_index, W1, b1, W2, b2):
    ei = edge_index.astype(jnp.int32)
    src, dst = ei[0], ei[1]
    pad = EP - E
    pad_ids = (jnp.arange(pad, dtype=jnp.int32) % (NP - N)) + N
    src_p = jnp.concatenate([src, pad_ids])
    dst_p = jnp.concatenate([dst, pad_ids])
    src_t = src_p.reshape(NS, C, LANE)
    src_t = jnp.stack([src_t, src_t + NP])
    dst_t = dst_p.reshape(NS, C, LANE)
    dst_w = dst_p.reshape(NC * NS, C // 2, LANE)

    x_pad = jnp.zeros((NP, D), jnp.float32).at[:N].set(x.astype(jnp.float32))

    dh = _deg(dst_w).reshape(NC, NP)
    y1 = _tc1(x_pad, W1, dh)
    s1 = _gather_scatter(y1.reshape(NC * NP, DH), src_t, dst_t)
    y2 = _tc2(s1, y1, dh, b1, W2)
    s2 = _gather_scatter(y2.reshape(NC * NP, DH), src_t, dst_t)
    return _tc3(s2, y2, dh, b2)
